# baseline (speedup 1.0000x reference)
import functools

import jax
import jax.numpy as jnp
from jax import lax
from jax.experimental import pallas as pl
from jax.experimental.pallas import tpu as pltpu
from jax.experimental.pallas import tpu_sc as plsc

N_NODES = 10000
N_PAD = 10240
ROWS_PER_TILE = N_PAD // 16
N_EDGES = 320000
CHUNK = 128
CH_PER_W = 80
N_WORKERS = 32
E_PAD = N_WORKERS * CH_PER_W * CHUNK
D = 128

_mesh = plsc.VectorSubcoreMesh(core_axis_name="c", subcore_axis_name="s")


def _zero_vmem_2d(ref, n_rows):
    z = jnp.zeros((16,), jnp.float32)

    def body(r, _):
        for k in range(8):
            ref[r, pl.ds(k * 16, 16)] = z
        return 0

    lax.fori_loop(0, n_rows, body, 0)


@functools.partial(
    pl.kernel,
    out_type=(
        jax.ShapeDtypeStruct((N_PAD,), jnp.float32),
        jax.ShapeDtypeStruct((N_PAD,), jnp.float32),
    ),
    mesh=_mesh,
    scratch_types=[
        pltpu.VMEM((CH_PER_W, CHUNK), jnp.int32),
        pltpu.VMEM((CHUNK,), jnp.float32),
        pltpu.VMEM((ROWS_PER_TILE,), jnp.float32),
        pltpu.VMEM_SHARED((N_PAD,), jnp.float32),
        pltpu.SemaphoreType.DMA,
    ],
)
def _hist(dst_hbm, out0_hbm, out1_hbm, dst_v, ones_v, zb_v, hist_sh, sem):
    c = lax.axis_index("c")
    s = lax.axis_index("s")
    w = c * 16 + s

    one = jnp.ones((16,), jnp.float32)
    zero = jnp.zeros((16,), jnp.float32)
    for k in range(8):
        ones_v[pl.ds(k * 16, 16)] = one

    def zb_body(i, _):
        zb_v[pl.ds(i * 16, 16)] = zero
        return 0

    lax.fori_loop(0, ROWS_PER_TILE // 16, zb_body, 0)
    pltpu.sync_copy(zb_v, hist_sh.at[pl.ds(s * ROWS_PER_TILE, ROWS_PER_TILE)])
    plsc.subcore_barrier()

    pltpu.sync_copy(dst_hbm.at[pl.ds(w * CH_PER_W, CH_PER_W)], dst_v)

    def body(j, _):
        pltpu.async_copy(ones_v, hist_sh.at[dst_v.at[j]], sem, add=True)

        @pl.when(j >= 8)
        def _():
            pltpu.make_async_copy(ones_v, hist_sh.at[dst_v.at[0]], sem).wait()

        return 0

    lax.fori_loop(0, CH_PER_W, body, 0)
    for _ in range(8):
        pltpu.make_async_copy(ones_v, hist_sh.at[dst_v.at[0]], sem).wait()
    plsc.subcore_barrier()

    @pl.when(c == 0)
    def _():
        pltpu.sync_copy(
            hist_sh.at[pl.ds(s * ROWS_PER_TILE, ROWS_PER_TILE)],
            out0_hbm.at[pl.ds(s * ROWS_PER_TILE, ROWS_PER_TILE)],
        )

    @pl.when(c == 1)
    def _():
        pltpu.sync_copy(
            hist_sh.at[pl.ds(s * ROWS_PER_TILE, ROWS_PER_TILE)],
            out1_hbm.at[pl.ds(s * ROWS_PER_TILE, ROWS_PER_TILE)],
        )


@functools.partial(
    pl.kernel,
    out_type=(
        jax.ShapeDtypeStruct((N_PAD, D), jnp.float32),
        jax.ShapeDtypeStruct((N_PAD, D), jnp.float32),
    ),
    mesh=_mesh,
    scratch_types=[
        pltpu.VMEM((CH_PER_W // 2, CHUNK), jnp.int32),
        pltpu.VMEM((CH_PER_W // 2, CHUNK), jnp.int32),
        pltpu.VMEM((CHUNK, D), jnp.float32),
        pltpu.VMEM((CHUNK, D), jnp.float32),
        pltpu.VMEM_SHARED((N_PAD, D), jnp.float32),
        pltpu.SemaphoreType.DMA,
        pltpu.SemaphoreType.DMA,
        pltpu.SemaphoreType.DMA,
        pltpu.SemaphoreType.DMA,
    ],
)
def _agg(y_hbm, src_hbm, dst_hbm, out0_hbm, out1_hbm, src_v, dst_v, rows_a, rows_b,
         acc_sh, gsem_a, gsem_b, ssem_a, ssem_b):
    c = lax.axis_index("c")
    s = lax.axis_index("s")
    w = c * 16 + s
    half_n = CH_PER_W // 2

    _zero_vmem_2d(rows_a, CHUNK)
    for i in range(ROWS_PER_TILE // CHUNK):
        pltpu.sync_copy(
            rows_a, acc_sh.at[pl.ds(s * ROWS_PER_TILE + i * CHUNK, CHUNK)]
        )
    plsc.subcore_barrier()

    def _gwait(rows, gsem):
        pltpu.make_async_copy(y_hbm.at[src_v.at[0]], rows, gsem).wait()

    def _swait(rows, ssem):
        pltpu.make_async_copy(rows, acc_sh.at[dst_v.at[0]], ssem).wait()

    for half in range(2):
        base = w * CH_PER_W + half * half_n
        pltpu.sync_copy(src_hbm.at[pl.ds(base, half_n)], src_v)
        pltpu.sync_copy(dst_hbm.at[pl.ds(base, half_n)], dst_v)
        pltpu.async_copy(y_hbm.at[src_v.at[0]], rows_a, gsem_a)
        pltpu.async_copy(y_hbm.at[src_v.at[1]], rows_b, gsem_b)

        def body(p, _):
            j = 2 * p
            _gwait(rows_a, gsem_a)
            pltpu.async_copy(rows_a, acc_sh.at[dst_v.at[j]], ssem_a, add=True)
            _gwait(rows_b, gsem_b)
            pltpu.async_copy(
                rows_b, acc_sh.at[dst_v.at[j + 1]], ssem_b, add=True
            )
            _swait(rows_a, ssem_a)

            @pl.when(p + 1 < half_n // 2)
            def _():
                pltpu.async_copy(y_hbm.at[src_v.at[j + 2]], rows_a, gsem_a)

            _swait(rows_b, ssem_b)

            @pl.when(p + 1 < half_n // 2)
            def _():
                pltpu.async_copy(y_hbm.at[src_v.at[j + 3]], rows_b, gsem_b)

            return 0

        lax.fori_loop(0, half_n // 2, body, 0)
    plsc.subcore_barrier()

    @pl.when(c == 0)
    def _():
        pltpu.sync_copy(
            acc_sh.at[pl.ds(s * ROWS_PER_TILE, ROWS_PER_TILE)],
            out0_hbm.at[pl.ds(s * ROWS_PER_TILE, ROWS_PER_TILE)],
        )

    @pl.when(c == 1)
    def _():
        pltpu.sync_copy(
            acc_sh.at[pl.ds(s * ROWS_PER_TILE, ROWS_PER_TILE)],
            out1_hbm.at[pl.ds(s * ROWS_PER_TILE, ROWS_PER_TILE)],
        )


_R = 400


def _mm_first_body(x_ref, w_ref, d0_ref, d1_ref, o_ref):
    deg = 1.0 + d0_ref[...] + d1_ref[...]
    dis = lax.rsqrt(deg)
    y = jnp.dot(x_ref[...], w_ref[...], preferred_element_type=jnp.float32)
    o_ref[...] = y * dis


def _mm_first(x, w, d0, d1):
    return pl.pallas_call(
        _mm_first_body,
        grid=(N_NODES // _R,),
        in_specs=[
            pl.BlockSpec((_R, D), lambda i: (i, 0)),
            pl.BlockSpec((D, D), lambda i: (0, 0)),
            pl.BlockSpec((_R, 1), lambda i: (i, 0)),
            pl.BlockSpec((_R, 1), lambda i: (i, 0)),
        ],
        out_specs=pl.BlockSpec((_R, D), lambda i: (i, 0)),
        out_shape=jax.ShapeDtypeStruct((N_NODES, D), jnp.float32),
    )(x, w, d0, d1)


def _mm_mid_body(a0_ref, a1_ref, y_ref, d0_ref, d1_ref, b_ref, w_ref, o_ref):
    deg = 1.0 + d0_ref[...] + d1_ref[...]
    dis = lax.rsqrt(deg)
    h = (a0_ref[...] + a1_ref[...] + y_ref[...]) * dis + b_ref[...]
    h = jnp.maximum(h, 0.0)
    o_ref[...] = (
        jnp.dot(h, w_ref[...], preferred_element_type=jnp.float32) * dis
    )


def _mm_mid(a0, a1, y, d0, d1, b, w):
    return pl.pallas_call(
        _mm_mid_body,
        grid=(N_NODES // _R,),
        in_specs=[
            pl.BlockSpec((_R, D), lambda i: (i, 0)),
            pl.BlockSpec((_R, D), lambda i: (i, 0)),
            pl.BlockSpec((_R, D), lambda i: (i, 0)),
            pl.BlockSpec((_R, 1), lambda i: (i, 0)),
            pl.BlockSpec((_R, 1), lambda i: (i, 0)),
            pl.BlockSpec((1, D), lambda i: (0, 0)),
            pl.BlockSpec((D, D), lambda i: (0, 0)),
        ],
        out_specs=pl.BlockSpec((_R, D), lambda i: (i, 0)),
        out_shape=jax.ShapeDtypeStruct((N_NODES, D), jnp.float32),
    )(a0, a1, y, d0, d1, b, w)


def _final_body(a0_ref, a1_ref, y_ref, d0_ref, d1_ref, b_ref, o_ref):
    deg = 1.0 + d0_ref[...] + d1_ref[...]
    dis = lax.rsqrt(deg)
    h = (a0_ref[...] + a1_ref[...] + y_ref[...]) * dis + b_ref[...]
    o_ref[...] = jnp.maximum(h, 0.0)


def _final(a0, a1, y, d0, d1, b):
    return pl.pallas_call(
        _final_body,
        grid=(N_NODES // _R,),
        in_specs=[
            pl.BlockSpec((_R, D), lambda i: (i, 0)),
            pl.BlockSpec((_R, D), lambda i: (i, 0)),
            pl.BlockSpec((_R, D), lambda i: (i, 0)),
            pl.BlockSpec((_R, 1), lambda i: (i, 0)),
            pl.BlockSpec((_R, 1), lambda i: (i, 0)),
            pl.BlockSpec((1, D), lambda i: (0, 0)),
        ],
        out_specs=pl.BlockSpec((_R, D), lambda i: (i, 0)),
        out_shape=jax.ShapeDtypeStruct((N_NODES, D), jnp.float32),
    )(a0, a1, y, d0, d1, b)


def kernel(mesh_node_features, edge_index, W1, b1, W2, b2, W3, b3):
    src = edge_index[0].astype(jnp.int32)
    dst = edge_index[1].astype(jnp.int32)
    pad = E_PAD - N_EDGES
    ar = jnp.arange(pad, dtype=jnp.int32)
    src_p = jnp.concatenate([src, ar % N_NODES]).reshape(
        N_WORKERS * CH_PER_W, CHUNK
    )
    dst_p = jnp.concatenate([dst, N_NODES + ar % (N_PAD - N_NODES)]).reshape(
        N_WORKERS * CH_PER_W, CHUNK
    )

    h0, h1 = _hist(dst_p)
    d0 = h0.reshape(N_PAD, 1)
    d1 = h1.reshape(N_PAD, 1)

    y1 = _mm_first(mesh_node_features, W1, d0, d1)
    a0, a1 = _agg(y1, src_p, dst_p)
    b1r = b1.reshape(1, D)
    b2r = b2.reshape(1, D)
    b3r = b3.reshape(1, D)
    y2 = _mm_mid(a0, a1, y1, d0, d1, b1r, W2)
    a0, a1 = _agg(y2, src_p, dst_p)
    y3 = _mm_mid(a0, a1, y2, d0, d1, b2r, W3)
    a0, a1 = _agg(y3, src_p, dst_p)
    return _final(a0, a1, y3, d0, d1, b3r)


# speedup vs baseline: 1.2462x; 1.2462x over previous
import functools

import jax
import jax.numpy as jnp
from jax import lax
from jax.experimental import pallas as pl
from jax.experimental.pallas import tpu as pltpu
from jax.experimental.pallas import tpu_sc as plsc

N_NODES = 10000
N_PAD = 10240
ROWS_PER_TILE = N_PAD // 16
N_EDGES = 320000
CHUNK = 128
CH_PER_W = 80
N_WORKERS = 32
E_PAD = N_WORKERS * CH_PER_W * CHUNK
D = 128

_mesh = plsc.VectorSubcoreMesh(core_axis_name="c", subcore_axis_name="s")


def _zero_vmem_2d(ref, n_rows):
    z = jnp.zeros((16,), jnp.float32)

    def body(r, _):
        for k in range(8):
            ref[r, pl.ds(k * 16, 16)] = z
        return 0

    lax.fori_loop(0, n_rows, body, 0)


@functools.partial(
    pl.kernel,
    out_type=(
        jax.ShapeDtypeStruct((N_PAD,), jnp.float32),
        jax.ShapeDtypeStruct((N_PAD,), jnp.float32),
    ),
    mesh=_mesh,
    scratch_types=[
        pltpu.VMEM((CH_PER_W, CHUNK), jnp.int32),
        pltpu.VMEM((CHUNK,), jnp.float32),
        pltpu.VMEM((ROWS_PER_TILE,), jnp.float32),
        pltpu.VMEM_SHARED((N_PAD,), jnp.float32),
        pltpu.SemaphoreType.DMA,
    ],
)
def _hist(dst_hbm, out0_hbm, out1_hbm, dst_v, ones_v, zb_v, hist_sh, sem):
    c = lax.axis_index("c")
    s = lax.axis_index("s")
    w = c * 16 + s

    one = jnp.ones((16,), jnp.float32)
    zero = jnp.zeros((16,), jnp.float32)
    for k in range(8):
        ones_v[pl.ds(k * 16, 16)] = one

    def zb_body(i, _):
        zb_v[pl.ds(i * 16, 16)] = zero
        return 0

    lax.fori_loop(0, ROWS_PER_TILE // 16, zb_body, 0)
    pltpu.sync_copy(zb_v, hist_sh.at[pl.ds(s * ROWS_PER_TILE, ROWS_PER_TILE)])
    plsc.subcore_barrier()

    pltpu.sync_copy(dst_hbm.at[pl.ds(w * CH_PER_W, CH_PER_W)], dst_v)

    def body(j, _):
        pltpu.async_copy(ones_v, hist_sh.at[dst_v.at[j]], sem, add=True)

        @pl.when(j >= 8)
        def _():
            pltpu.make_async_copy(ones_v, hist_sh.at[dst_v.at[0]], sem).wait()

        return 0

    lax.fori_loop(0, CH_PER_W, body, 0)
    for _ in range(8):
        pltpu.make_async_copy(ones_v, hist_sh.at[dst_v.at[0]], sem).wait()
    plsc.subcore_barrier()

    @pl.when(c == 0)
    def _():
        pltpu.sync_copy(
            hist_sh.at[pl.ds(s * ROWS_PER_TILE, ROWS_PER_TILE)],
            out0_hbm.at[pl.ds(s * ROWS_PER_TILE, ROWS_PER_TILE)],
        )

    @pl.when(c == 1)
    def _():
        pltpu.sync_copy(
            hist_sh.at[pl.ds(s * ROWS_PER_TILE, ROWS_PER_TILE)],
            out1_hbm.at[pl.ds(s * ROWS_PER_TILE, ROWS_PER_TILE)],
        )


@functools.partial(
    pl.kernel,
    out_type=(
        jax.ShapeDtypeStruct((N_PAD, D), jnp.float32),
        jax.ShapeDtypeStruct((N_PAD, D), jnp.float32),
    ),
    mesh=_mesh,
    scratch_types=[
        pltpu.VMEM((CH_PER_W // 2, CHUNK), jnp.int32),
        pltpu.VMEM((CH_PER_W // 2, CHUNK), jnp.int32),
        pltpu.VMEM((CHUNK, D), jnp.float32),
        pltpu.VMEM((CHUNK, D), jnp.float32),
        pltpu.VMEM_SHARED((N_PAD, D), jnp.float32),
        pltpu.SemaphoreType.DMA,
        pltpu.SemaphoreType.DMA,
    ],
)
def _agg(y_hbm, src_hbm, dst_hbm, out0_hbm, out1_hbm, src_v, dst_v, rows_a, rows_b,
         acc_sh, gsem_a, gsem_b):
    c = lax.axis_index("c")
    s = lax.axis_index("s")
    w = c * 16 + s
    half_n = CH_PER_W // 2

    _zero_vmem_2d(rows_a, CHUNK)
    for i in range(ROWS_PER_TILE // CHUNK):
        pltpu.sync_copy(
            rows_a, acc_sh.at[pl.ds(s * ROWS_PER_TILE + i * CHUNK, CHUNK)]
        )
    plsc.subcore_barrier()

    def _gwait(rows, gsem):
        pltpu.make_async_copy(y_hbm.at[src_v.at[0]], rows, gsem).wait()

    for half in range(2):
        base = w * CH_PER_W + half * half_n
        pltpu.sync_copy(src_hbm.at[pl.ds(base, half_n)], src_v)
        pltpu.sync_copy(dst_hbm.at[pl.ds(base, half_n)], dst_v)
        pltpu.async_copy(y_hbm.at[src_v.at[0]], rows_a, gsem_a)
        pltpu.async_copy(y_hbm.at[src_v.at[1]], rows_b, gsem_b)

        def body(p, _):
            j = 2 * p
            _gwait(rows_a, gsem_a)
            pltpu.sync_copy(rows_a, acc_sh.at[dst_v.at[j]], add=True)

            @pl.when(p + 1 < half_n // 2)
            def _():
                pltpu.async_copy(y_hbm.at[src_v.at[j + 2]], rows_a, gsem_a)

            _gwait(rows_b, gsem_b)
            pltpu.sync_copy(rows_b, acc_sh.at[dst_v.at[j + 1]], add=True)

            @pl.when(p + 1 < half_n // 2)
            def _():
                pltpu.async_copy(y_hbm.at[src_v.at[j + 3]], rows_b, gsem_b)

            return 0

        lax.fori_loop(0, half_n // 2, body, 0)
    plsc.subcore_barrier()

    @pl.when(c == 0)
    def _():
        pltpu.sync_copy(
            acc_sh.at[pl.ds(s * ROWS_PER_TILE, ROWS_PER_TILE)],
            out0_hbm.at[pl.ds(s * ROWS_PER_TILE, ROWS_PER_TILE)],
        )

    @pl.when(c == 1)
    def _():
        pltpu.sync_copy(
            acc_sh.at[pl.ds(s * ROWS_PER_TILE, ROWS_PER_TILE)],
            out1_hbm.at[pl.ds(s * ROWS_PER_TILE, ROWS_PER_TILE)],
        )


_R = 400


def _mm_first_body(x_ref, w_ref, d0_ref, d1_ref, o_ref):
    deg = 1.0 + d0_ref[...] + d1_ref[...]
    dis = lax.rsqrt(deg)
    y = jnp.dot(x_ref[...], w_ref[...], preferred_element_type=jnp.float32)
    o_ref[...] = y * dis


def _mm_first(x, w, d0, d1):
    return pl.pallas_call(
        _mm_first_body,
        grid=(N_NODES // _R,),
        in_specs=[
            pl.BlockSpec((_R, D), lambda i: (i, 0)),
            pl.BlockSpec((D, D), lambda i: (0, 0)),
            pl.BlockSpec((_R, 1), lambda i: (i, 0)),
            pl.BlockSpec((_R, 1), lambda i: (i, 0)),
        ],
        out_specs=pl.BlockSpec((_R, D), lambda i: (i, 0)),
        out_shape=jax.ShapeDtypeStruct((N_NODES, D), jnp.float32),
    )(x, w, d0, d1)


def _mm_mid_body(a0_ref, a1_ref, y_ref, d0_ref, d1_ref, b_ref, w_ref, o_ref):
    deg = 1.0 + d0_ref[...] + d1_ref[...]
    dis = lax.rsqrt(deg)
    h = (a0_ref[...] + a1_ref[...] + y_ref[...]) * dis + b_ref[...]
    h = jnp.maximum(h, 0.0)
    o_ref[...] = (
        jnp.dot(h, w_ref[...], preferred_element_type=jnp.float32) * dis
    )


def _mm_mid(a0, a1, y, d0, d1, b, w):
    return pl.pallas_call(
        _mm_mid_body,
        grid=(N_NODES // _R,),
        in_specs=[
            pl.BlockSpec((_R, D), lambda i: (i, 0)),
            pl.BlockSpec((_R, D), lambda i: (i, 0)),
            pl.BlockSpec((_R, D), lambda i: (i, 0)),
            pl.BlockSpec((_R, 1), lambda i: (i, 0)),
            pl.BlockSpec((_R, 1), lambda i: (i, 0)),
            pl.BlockSpec((1, D), lambda i: (0, 0)),
            pl.BlockSpec((D, D), lambda i: (0, 0)),
        ],
        out_specs=pl.BlockSpec((_R, D), lambda i: (i, 0)),
        out_shape=jax.ShapeDtypeStruct((N_NODES, D), jnp.float32),
    )(a0, a1, y, d0, d1, b, w)


def _final_body(a0_ref, a1_ref, y_ref, d0_ref, d1_ref, b_ref, o_ref):
    deg = 1.0 + d0_ref[...] + d1_ref[...]
    dis = lax.rsqrt(deg)
    h = (a0_ref[...] + a1_ref[...] + y_ref[...]) * dis + b_ref[...]
    o_ref[...] = jnp.maximum(h, 0.0)


def _final(a0, a1, y, d0, d1, b):
    return pl.pallas_call(
        _final_body,
        grid=(N_NODES // _R,),
        in_specs=[
            pl.BlockSpec((_R, D), lambda i: (i, 0)),
            pl.BlockSpec((_R, D), lambda i: (i, 0)),
            pl.BlockSpec((_R, D), lambda i: (i, 0)),
            pl.BlockSpec((_R, 1), lambda i: (i, 0)),
            pl.BlockSpec((_R, 1), lambda i: (i, 0)),
            pl.BlockSpec((1, D), lambda i: (0, 0)),
        ],
        out_specs=pl.BlockSpec((_R, D), lambda i: (i, 0)),
        out_shape=jax.ShapeDtypeStruct((N_NODES, D), jnp.float32),
    )(a0, a1, y, d0, d1, b)


def kernel(mesh_node_features, edge_index, W1, b1, W2, b2, W3, b3):
    src = edge_index[0].astype(jnp.int32)
    dst = edge_index[1].astype(jnp.int32)
    pad = E_PAD - N_EDGES
    ar = jnp.arange(pad, dtype=jnp.int32)
    src_p = jnp.concatenate([src, ar % N_NODES]).reshape(
        N_WORKERS * CH_PER_W, CHUNK
    )
    dst_p = jnp.concatenate([dst, N_NODES + ar % (N_PAD - N_NODES)]).reshape(
        N_WORKERS * CH_PER_W, CHUNK
    )

    h0, h1 = _hist(dst_p)
    d0 = h0.reshape(N_PAD, 1)
    d1 = h1.reshape(N_PAD, 1)

    y1 = _mm_first(mesh_node_features, W1, d0, d1)
    a0, a1 = _agg(y1, src_p, dst_p)
    b1r = b1.reshape(1, D)
    b2r = b2.reshape(1, D)
    b3r = b3.reshape(1, D)
    y2 = _mm_mid(a0, a1, y1, d0, d1, b1r, W2)
    a0, a1 = _agg(y2, src_p, dst_p)
    y3 = _mm_mid(a0, a1, y2, d0, d1, b2r, W3)
    a0, a1 = _agg(y3, src_p, dst_p)
    return _final(a0, a1, y3, d0, d1, b3r)


# speedup vs baseline: 1.3683x; 1.0979x over previous
import functools

import numpy as np

import jax
import jax.numpy as jnp
from jax import lax
from jax.experimental import pallas as pl
from jax.experimental.pallas import tpu as pltpu
from jax.experimental.pallas import tpu_sc as plsc

N_NODES = 10000
N_PAD = 10240
ROWS_PER_TILE = N_PAD // 16
N_EDGES = 320000
CHUNK = 128
CH_PER_W = 80
N_WORKERS = 32
E_PAD = N_WORKERS * CH_PER_W * CHUNK
D = 128

_mesh = plsc.VectorSubcoreMesh(core_axis_name="c", subcore_axis_name="s")

_PAD_N = E_PAD - N_EDGES
_SRC_PAD = (np.arange(_PAD_N) % N_NODES).astype(np.int32)
_DST_PAD = (N_NODES + np.arange(_PAD_N) % (N_PAD - N_NODES)).astype(np.int32)


def _zero_vmem_2d(ref, n_rows):
    z = jnp.zeros((16,), jnp.float32)

    def body(r, _):
        for k in range(8):
            ref[r, pl.ds(k * 16, 16)] = z
        return 0

    lax.fori_loop(0, n_rows, body, 0)


@functools.partial(
    pl.kernel,
    out_type=(
        jax.ShapeDtypeStruct((N_PAD,), jnp.float32),
        jax.ShapeDtypeStruct((N_PAD,), jnp.float32),
    ),
    mesh=_mesh,
    scratch_types=[
        pltpu.VMEM((CH_PER_W, CHUNK), jnp.int32),
        pltpu.VMEM((CHUNK,), jnp.float32),
        pltpu.VMEM((ROWS_PER_TILE,), jnp.float32),
        pltpu.VMEM_SHARED((N_PAD,), jnp.float32),
        pltpu.SemaphoreType.DMA,
    ],
)
def _hist(dst_hbm, out0_hbm, out1_hbm, dst_v, ones_v, zb_v, hist_sh, sem):
    c = lax.axis_index("c")
    s = lax.axis_index("s")
    w = c * 16 + s

    one = jnp.ones((16,), jnp.float32)
    zero = jnp.zeros((16,), jnp.float32)
    for k in range(8):
        ones_v[pl.ds(k * 16, 16)] = one

    def zb_body(i, _):
        zb_v[pl.ds(i * 16, 16)] = zero
        return 0

    lax.fori_loop(0, ROWS_PER_TILE // 16, zb_body, 0)
    pltpu.sync_copy(zb_v, hist_sh.at[pl.ds(s * ROWS_PER_TILE, ROWS_PER_TILE)])
    plsc.subcore_barrier()

    pltpu.sync_copy(dst_hbm.at[pl.ds(w * CH_PER_W, CH_PER_W)], dst_v)

    def body(j, _):
        pltpu.async_copy(ones_v, hist_sh.at[dst_v.at[j]], sem, add=True)

        @pl.when(j >= 8)
        def _():
            pltpu.make_async_copy(ones_v, hist_sh.at[dst_v.at[0]], sem).wait()

        return 0

    lax.fori_loop(0, CH_PER_W, body, 0)
    for _ in range(8):
        pltpu.make_async_copy(ones_v, hist_sh.at[dst_v.at[0]], sem).wait()
    plsc.subcore_barrier()

    @pl.when(c == 0)
    def _():
        pltpu.sync_copy(
            hist_sh.at[pl.ds(s * ROWS_PER_TILE, ROWS_PER_TILE)],
            out0_hbm.at[pl.ds(s * ROWS_PER_TILE, ROWS_PER_TILE)],
        )

    @pl.when(c == 1)
    def _():
        pltpu.sync_copy(
            hist_sh.at[pl.ds(s * ROWS_PER_TILE, ROWS_PER_TILE)],
            out1_hbm.at[pl.ds(s * ROWS_PER_TILE, ROWS_PER_TILE)],
        )


@functools.partial(
    pl.kernel,
    out_type=(
        jax.ShapeDtypeStruct((N_PAD, D), jnp.float32),
        jax.ShapeDtypeStruct((N_PAD, D), jnp.float32),
    ),
    mesh=_mesh,
    scratch_types=[
        pltpu.VMEM((CH_PER_W // 2, CHUNK), jnp.int32),
        pltpu.VMEM((CH_PER_W // 2, CHUNK), jnp.int32),
        pltpu.VMEM((CHUNK, D), jnp.float32),
        pltpu.VMEM((CHUNK, D), jnp.float32),
        pltpu.VMEM_SHARED((N_PAD, D), jnp.float32),
        pltpu.SemaphoreType.DMA,
        pltpu.SemaphoreType.DMA,
    ],
)
def _agg(y_hbm, src_hbm, dst_hbm, out0_hbm, out1_hbm, src_v, dst_v, rows_a, rows_b,
         acc_sh, gsem_a, gsem_b):
    c = lax.axis_index("c")
    s = lax.axis_index("s")
    w = c * 16 + s
    half_n = CH_PER_W // 2

    _zero_vmem_2d(rows_a, CHUNK)
    for i in range(ROWS_PER_TILE // CHUNK):
        pltpu.sync_copy(
            rows_a, acc_sh.at[pl.ds(s * ROWS_PER_TILE + i * CHUNK, CHUNK)]
        )
    plsc.subcore_barrier()

    def _gwait(rows, gsem):
        pltpu.make_async_copy(y_hbm.at[src_v.at[0]], rows, gsem).wait()

    for half in range(2):
        base = w * CH_PER_W + half * half_n
        pltpu.sync_copy(src_hbm.at[pl.ds(base, half_n)], src_v)
        pltpu.sync_copy(dst_hbm.at[pl.ds(base, half_n)], dst_v)
        pltpu.async_copy(y_hbm.at[src_v.at[0]], rows_a, gsem_a)
        pltpu.async_copy(y_hbm.at[src_v.at[1]], rows_b, gsem_b)

        def body(p, _):
            j = 2 * p
            _gwait(rows_a, gsem_a)
            pltpu.sync_copy(rows_a, acc_sh.at[dst_v.at[j]], add=True)

            @pl.when(p + 1 < half_n // 2)
            def _():
                pltpu.async_copy(y_hbm.at[src_v.at[j + 2]], rows_a, gsem_a)

            _gwait(rows_b, gsem_b)
            pltpu.sync_copy(rows_b, acc_sh.at[dst_v.at[j + 1]], add=True)

            @pl.when(p + 1 < half_n // 2)
            def _():
                pltpu.async_copy(y_hbm.at[src_v.at[j + 3]], rows_b, gsem_b)

            return 0

        lax.fori_loop(0, half_n // 2, body, 0)
    plsc.subcore_barrier()

    @pl.when(c == 0)
    def _():
        pltpu.sync_copy(
            acc_sh.at[pl.ds(s * ROWS_PER_TILE, ROWS_PER_TILE)],
            out0_hbm.at[pl.ds(s * ROWS_PER_TILE, ROWS_PER_TILE)],
        )

    @pl.when(c == 1)
    def _():
        pltpu.sync_copy(
            acc_sh.at[pl.ds(s * ROWS_PER_TILE, ROWS_PER_TILE)],
            out1_hbm.at[pl.ds(s * ROWS_PER_TILE, ROWS_PER_TILE)],
        )


_R = 2000


def _mm_first_body(x_ref, w_ref, d0_ref, d1_ref, o_ref):
    deg = 1.0 + d0_ref[...] + d1_ref[...]
    dis = lax.rsqrt(deg)
    y = jnp.dot(x_ref[...], w_ref[...], preferred_element_type=jnp.float32)
    o_ref[...] = y * dis


def _mm_first(x, w, d0, d1):
    return pl.pallas_call(
        _mm_first_body,
        grid=(N_NODES // _R,),
        in_specs=[
            pl.BlockSpec((_R, D), lambda i: (i, 0)),
            pl.BlockSpec((D, D), lambda i: (0, 0)),
            pl.BlockSpec((_R, 1), lambda i: (i, 0)),
            pl.BlockSpec((_R, 1), lambda i: (i, 0)),
        ],
        out_specs=pl.BlockSpec((_R, D), lambda i: (i, 0)),
        out_shape=jax.ShapeDtypeStruct((N_NODES, D), jnp.float32),
    )(x, w, d0, d1)


def _mm_mid_body(a0_ref, a1_ref, y_ref, d0_ref, d1_ref, b_ref, w_ref, o_ref):
    deg = 1.0 + d0_ref[...] + d1_ref[...]
    dis = lax.rsqrt(deg)
    h = (a0_ref[...] + a1_ref[...] + y_ref[...]) * dis + b_ref[...]
    h = jnp.maximum(h, 0.0)
    o_ref[...] = (
        jnp.dot(h, w_ref[...], preferred_element_type=jnp.float32) * dis
    )


def _mm_mid(a0, a1, y, d0, d1, b, w):
    return pl.pallas_call(
        _mm_mid_body,
        grid=(N_NODES // _R,),
        in_specs=[
            pl.BlockSpec((_R, D), lambda i: (i, 0)),
            pl.BlockSpec((_R, D), lambda i: (i, 0)),
            pl.BlockSpec((_R, D), lambda i: (i, 0)),
            pl.BlockSpec((_R, 1), lambda i: (i, 0)),
            pl.BlockSpec((_R, 1), lambda i: (i, 0)),
            pl.BlockSpec((1, D), lambda i: (0, 0)),
            pl.BlockSpec((D, D), lambda i: (0, 0)),
        ],
        out_specs=pl.BlockSpec((_R, D), lambda i: (i, 0)),
        out_shape=jax.ShapeDtypeStruct((N_NODES, D), jnp.float32),
    )(a0, a1, y, d0, d1, b, w)


def _final_body(a0_ref, a1_ref, y_ref, d0_ref, d1_ref, b_ref, o_ref):
    deg = 1.0 + d0_ref[...] + d1_ref[...]
    dis = lax.rsqrt(deg)
    h = (a0_ref[...] + a1_ref[...] + y_ref[...]) * dis + b_ref[...]
    o_ref[...] = jnp.maximum(h, 0.0)


def _final(a0, a1, y, d0, d1, b):
    return pl.pallas_call(
        _final_body,
        grid=(N_NODES // _R,),
        in_specs=[
            pl.BlockSpec((_R, D), lambda i: (i, 0)),
            pl.BlockSpec((_R, D), lambda i: (i, 0)),
            pl.BlockSpec((_R, D), lambda i: (i, 0)),
            pl.BlockSpec((_R, 1), lambda i: (i, 0)),
            pl.BlockSpec((_R, 1), lambda i: (i, 0)),
            pl.BlockSpec((1, D), lambda i: (0, 0)),
        ],
        out_specs=pl.BlockSpec((_R, D), lambda i: (i, 0)),
        out_shape=jax.ShapeDtypeStruct((N_NODES, D), jnp.float32),
    )(a0, a1, y, d0, d1, b)


def kernel(mesh_node_features, edge_index, W1, b1, W2, b2, W3, b3):
    src = edge_index[0].astype(jnp.int32)
    dst = edge_index[1].astype(jnp.int32)
    src_p = jnp.concatenate([src, _SRC_PAD]).reshape(
        N_WORKERS * CH_PER_W, CHUNK
    )
    dst_p = jnp.concatenate([dst, _DST_PAD]).reshape(
        N_WORKERS * CH_PER_W, CHUNK
    )

    h0, h1 = _hist(dst_p)
    d0 = h0.reshape(N_PAD, 1)
    d1 = h1.reshape(N_PAD, 1)

    y1 = _mm_first(mesh_node_features, W1, d0, d1)
    a0, a1 = _agg(y1, src_p, dst_p)
    b1r = b1.reshape(1, D)
    b2r = b2.reshape(1, D)
    b3r = b3.reshape(1, D)
    y2 = _mm_mid(a0, a1, y1, d0, d1, b1r, W2)
    a0, a1 = _agg(y2, src_p, dst_p)
    y3 = _mm_mid(a0, a1, y2, d0, d1, b2r, W3)
    a0, a1 = _agg(y3, src_p, dst_p)
    return _final(a0, a1, y3, d0, d1, b3r)


# speedup vs baseline: 1.3897x; 1.0157x over previous
import functools

import numpy as np

import jax
import jax.numpy as jnp
from jax import lax
from jax.experimental import pallas as pl
from jax.experimental.pallas import tpu as pltpu
from jax.experimental.pallas import tpu_sc as plsc

N_NODES = 10000
N_PAD = 10240
ROWS_PER_TILE = N_PAD // 16
N_EDGES = 320000
CHUNK = 128
CH_PER_W = 80
N_WORKERS = 32
D = 128
CH_MAIN = 2496
E_MAIN = CH_MAIN * CHUNK
CH_TAIL = (N_EDGES - E_MAIN) // CHUNK

_mesh = plsc.VectorSubcoreMesh(core_axis_name="c", subcore_axis_name="s")


def _zero_vmem_2d(ref, n_rows):
    z = jnp.zeros((16,), jnp.float32)

    def body(r, _):
        for k in range(8):
            ref[r, pl.ds(k * 16, 16)] = z
        return 0

    lax.fori_loop(0, n_rows, body, 0)


@functools.partial(
    pl.kernel,
    out_type=(
        jax.ShapeDtypeStruct((N_PAD,), jnp.float32),
        jax.ShapeDtypeStruct((N_PAD,), jnp.float32),
    ),
    mesh=_mesh,
    scratch_types=[
        pltpu.VMEM((CH_PER_W, CHUNK), jnp.int32),
        pltpu.VMEM((CHUNK,), jnp.float32),
        pltpu.VMEM((ROWS_PER_TILE,), jnp.float32),
        pltpu.VMEM_SHARED((N_PAD,), jnp.float32),
        pltpu.SemaphoreType.DMA,
    ],
)
def _hist(dst_hbm, dstt_hbm, out0_hbm, out1_hbm, dst_v, ones_v, zb_v,
          hist_sh, sem):
    c = lax.axis_index("c")
    s = lax.axis_index("s")
    w = c * 16 + s
    last = w == N_WORKERS - 1
    n_ch = jnp.where(last, 16 + CH_TAIL, CH_PER_W)

    one = jnp.ones((16,), jnp.float32)
    zero = jnp.zeros((16,), jnp.float32)
    for k in range(8):
        ones_v[pl.ds(k * 16, 16)] = one

    def zb_body(i, _):
        zb_v[pl.ds(i * 16, 16)] = zero
        return 0

    lax.fori_loop(0, ROWS_PER_TILE // 16, zb_body, 0)
    pltpu.sync_copy(zb_v, hist_sh.at[pl.ds(s * ROWS_PER_TILE, ROWS_PER_TILE)])
    plsc.subcore_barrier()

    @pl.when(jnp.logical_not(last))
    def _():
        pltpu.sync_copy(dst_hbm.at[pl.ds(w * CH_PER_W, CH_PER_W)], dst_v)

    @pl.when(last)
    def _():
        pltpu.sync_copy(
            dst_hbm.at[pl.ds(CH_MAIN - 16, 16)], dst_v.at[pl.ds(0, 16)]
        )
        pltpu.sync_copy(dstt_hbm, dst_v.at[pl.ds(16, CH_TAIL)])

    def body(j, _):
        pltpu.async_copy(ones_v, hist_sh.at[dst_v.at[j]], sem, add=True)

        @pl.when(j >= 8)
        def _():
            pltpu.make_async_copy(ones_v, hist_sh.at[dst_v.at[0]], sem).wait()

        return 0

    lax.fori_loop(0, n_ch, body, 0)
    for _ in range(8):
        pltpu.make_async_copy(ones_v, hist_sh.at[dst_v.at[0]], sem).wait()
    plsc.subcore_barrier()

    @pl.when(c == 0)
    def _():
        pltpu.sync_copy(
            hist_sh.at[pl.ds(s * ROWS_PER_TILE, ROWS_PER_TILE)],
            out0_hbm.at[pl.ds(s * ROWS_PER_TILE, ROWS_PER_TILE)],
        )

    @pl.when(c == 1)
    def _():
        pltpu.sync_copy(
            hist_sh.at[pl.ds(s * ROWS_PER_TILE, ROWS_PER_TILE)],
            out1_hbm.at[pl.ds(s * ROWS_PER_TILE, ROWS_PER_TILE)],
        )


@functools.partial(
    pl.kernel,
    out_type=(
        jax.ShapeDtypeStruct((N_PAD, D), jnp.float32),
        jax.ShapeDtypeStruct((N_PAD, D), jnp.float32),
    ),
    mesh=_mesh,
    scratch_types=[
        pltpu.VMEM((CH_PER_W // 2, CHUNK), jnp.int32),
        pltpu.VMEM((CH_PER_W // 2, CHUNK), jnp.int32),
        pltpu.VMEM((CHUNK, D), jnp.float32),
        pltpu.VMEM((CHUNK, D), jnp.float32),
        pltpu.VMEM_SHARED((N_PAD, D), jnp.float32),
        pltpu.SemaphoreType.DMA,
        pltpu.SemaphoreType.DMA,
    ],
)
def _agg(y_hbm, src_hbm, dst_hbm, srct_hbm, dstt_hbm, out0_hbm, out1_hbm,
         src_v, dst_v, rows_a, rows_b, acc_sh, gsem_a, gsem_b):
    c = lax.axis_index("c")
    s = lax.axis_index("s")
    w = c * 16 + s
    last = w == N_WORKERS - 1
    half_n = CH_PER_W // 2

    _zero_vmem_2d(rows_a, CHUNK)
    for i in range(ROWS_PER_TILE // CHUNK):
        pltpu.sync_copy(
            rows_a, acc_sh.at[pl.ds(s * ROWS_PER_TILE + i * CHUNK, CHUNK)]
        )
    plsc.subcore_barrier()

    def _gwait(rows, gsem):
        pltpu.make_async_copy(y_hbm.at[src_v.at[0]], rows, gsem).wait()

    for half in range(2):
        base = w * CH_PER_W + half * half_n
        if half == 0:
            n_pairs = jnp.where(last, (16 + CH_TAIL) // 2, half_n // 2)
            active = w >= 0

            @pl.when(jnp.logical_not(last))
            def _():
                pltpu.sync_copy(src_hbm.at[pl.ds(base, half_n)], src_v)
                pltpu.sync_copy(dst_hbm.at[pl.ds(base, half_n)], dst_v)

            @pl.when(last)
            def _():
                pltpu.sync_copy(
                    src_hbm.at[pl.ds(CH_MAIN - 16, 16)],
                    src_v.at[pl.ds(0, 16)],
                )
                pltpu.sync_copy(
                    dst_hbm.at[pl.ds(CH_MAIN - 16, 16)],
                    dst_v.at[pl.ds(0, 16)],
                )
                pltpu.sync_copy(srct_hbm, src_v.at[pl.ds(16, CH_TAIL)])
                pltpu.sync_copy(dstt_hbm, dst_v.at[pl.ds(16, CH_TAIL)])
        else:
            n_pairs = jnp.where(last, 0, half_n // 2)
            active = jnp.logical_not(last)

            @pl.when(active)
            def _():
                pltpu.sync_copy(src_hbm.at[pl.ds(base, half_n)], src_v)
                pltpu.sync_copy(dst_hbm.at[pl.ds(base, half_n)], dst_v)

        @pl.when(active)
        def _():
            pltpu.async_copy(y_hbm.at[src_v.at[0]], rows_a, gsem_a)
            pltpu.async_copy(y_hbm.at[src_v.at[1]], rows_b, gsem_b)

        def body(p, _):
            j = 2 * p
            _gwait(rows_a, gsem_a)
            pltpu.sync_copy(rows_a, acc_sh.at[dst_v.at[j]], add=True)

            @pl.when(p + 1 < n_pairs)
            def _():
                pltpu.async_copy(y_hbm.at[src_v.at[j + 2]], rows_a, gsem_a)

            _gwait(rows_b, gsem_b)
            pltpu.sync_copy(rows_b, acc_sh.at[dst_v.at[j + 1]], add=True)

            @pl.when(p + 1 < n_pairs)
            def _():
                pltpu.async_copy(y_hbm.at[src_v.at[j + 3]], rows_b, gsem_b)

            return 0

        lax.fori_loop(0, n_pairs, body, 0)
    plsc.subcore_barrier()

    @pl.when(c == 0)
    def _():
        pltpu.sync_copy(
            acc_sh.at[pl.ds(s * ROWS_PER_TILE, ROWS_PER_TILE)],
            out0_hbm.at[pl.ds(s * ROWS_PER_TILE, ROWS_PER_TILE)],
        )

    @pl.when(c == 1)
    def _():
        pltpu.sync_copy(
            acc_sh.at[pl.ds(s * ROWS_PER_TILE, ROWS_PER_TILE)],
            out1_hbm.at[pl.ds(s * ROWS_PER_TILE, ROWS_PER_TILE)],
        )


_R = 2000


def _mm_first_body(x_ref, w_ref, d0_ref, d1_ref, o_ref):
    deg = 1.0 + d0_ref[...] + d1_ref[...]
    dis = lax.rsqrt(deg)
    y = jnp.dot(x_ref[...], w_ref[...], preferred_element_type=jnp.float32)
    o_ref[...] = y * dis


def _mm_first(x, w, d0, d1):
    return pl.pallas_call(
        _mm_first_body,
        grid=(N_NODES // _R,),
        in_specs=[
            pl.BlockSpec((_R, D), lambda i: (i, 0)),
            pl.BlockSpec((D, D), lambda i: (0, 0)),
            pl.BlockSpec((_R, 1), lambda i: (i, 0)),
            pl.BlockSpec((_R, 1), lambda i: (i, 0)),
        ],
        out_specs=pl.BlockSpec((_R, D), lambda i: (i, 0)),
        out_shape=jax.ShapeDtypeStruct((N_NODES, D), jnp.float32),
    )(x, w, d0, d1)


def _mm_mid_body(a0_ref, a1_ref, y_ref, d0_ref, d1_ref, b_ref, w_ref, o_ref):
    deg = 1.0 + d0_ref[...] + d1_ref[...]
    dis = lax.rsqrt(deg)
    h = (a0_ref[...] + a1_ref[...] + y_ref[...]) * dis + b_ref[...]
    h = jnp.maximum(h, 0.0)
    o_ref[...] = (
        jnp.dot(h, w_ref[...], preferred_element_type=jnp.float32) * dis
    )


def _mm_mid(a0, a1, y, d0, d1, b, w):
    return pl.pallas_call(
        _mm_mid_body,
        grid=(N_NODES // _R,),
        in_specs=[
            pl.BlockSpec((_R, D), lambda i: (i, 0)),
            pl.BlockSpec((_R, D), lambda i: (i, 0)),
            pl.BlockSpec((_R, D), lambda i: (i, 0)),
            pl.BlockSpec((_R, 1), lambda i: (i, 0)),
            pl.BlockSpec((_R, 1), lambda i: (i, 0)),
            pl.BlockSpec((1, D), lambda i: (0, 0)),
            pl.BlockSpec((D, D), lambda i: (0, 0)),
        ],
        out_specs=pl.BlockSpec((_R, D), lambda i: (i, 0)),
        out_shape=jax.ShapeDtypeStruct((N_NODES, D), jnp.float32),
    )(a0, a1, y, d0, d1, b, w)


def _final_body(a0_ref, a1_ref, y_ref, d0_ref, d1_ref, b_ref, o_ref):
    deg = 1.0 + d0_ref[...] + d1_ref[...]
    dis = lax.rsqrt(deg)
    h = (a0_ref[...] + a1_ref[...] + y_ref[...]) * dis + b_ref[...]
    o_ref[...] = jnp.maximum(h, 0.0)


def _final(a0, a1, y, d0, d1, b):
    return pl.pallas_call(
        _final_body,
        grid=(N_NODES // _R,),
        in_specs=[
            pl.BlockSpec((_R, D), lambda i: (i, 0)),
            pl.BlockSpec((_R, D), lambda i: (i, 0)),
            pl.BlockSpec((_R, D), lambda i: (i, 0)),
            pl.BlockSpec((_R, 1), lambda i: (i, 0)),
            pl.BlockSpec((_R, 1), lambda i: (i, 0)),
            pl.BlockSpec((1, D), lambda i: (0, 0)),
        ],
        out_specs=pl.BlockSpec((_R, D), lambda i: (i, 0)),
        out_shape=jax.ShapeDtypeStruct((N_NODES, D), jnp.float32),
    )(a0, a1, y, d0, d1, b)


def kernel(mesh_node_features, edge_index, W1, b1, W2, b2, W3, b3):
    ei = edge_index.astype(jnp.int32)
    src_m = ei[0, :E_MAIN].reshape(CH_MAIN, CHUNK)
    dst_m = ei[1, :E_MAIN].reshape(CH_MAIN, CHUNK)
    src_t = ei[0, E_MAIN:].reshape(CH_TAIL, CHUNK)
    dst_t = ei[1, E_MAIN:].reshape(CH_TAIL, CHUNK)

    h0, h1 = _hist(dst_m, dst_t)
    d0 = h0.reshape(N_PAD, 1)
    d1 = h1.reshape(N_PAD, 1)

    y1 = _mm_first(mesh_node_features, W1, d0, d1)
    a0, a1 = _agg(y1, src_m, dst_m, src_t, dst_t)
    b1r = b1.reshape(1, D)
    b2r = b2.reshape(1, D)
    b3r = b3.reshape(1, D)
    y2 = _mm_mid(a0, a1, y1, d0, d1, b1r, W2)
    a0, a1 = _agg(y2, src_m, dst_m, src_t, dst_t)
    y3 = _mm_mid(a0, a1, y2, d0, d1, b2r, W3)
    a0, a1 = _agg(y3, src_m, dst_m, src_t, dst_t)
    return _final(a0, a1, y3, d0, d1, b3r)
